# Initial kernel scaffold; baseline (speedup 1.0000x reference)
import jax
import jax.numpy as jnp
from jax.experimental import pallas as pl


def kernel(shape_id, colour_id, edge_index, batch, shape_emb, col_emb, W1l, b1l, W1r, bn1_g, bn1_b, W2l, b2l, W2r, bn2_g, bn2_b, lin_W, lin_b):
    raise NotImplementedError("write your pallas kernel here")


# speedup vs baseline: 6.4722x; 6.4722x over previous
import functools

import jax
import jax.numpy as jnp
from jax import lax
from jax.experimental import pallas as pl
from jax.experimental.pallas import tpu as pltpu
from jax.experimental.pallas import tpu_sc as plsc

_N = 50000
_E = 800000
_H = 64
_NG = 512
_NS = 64
_NC = 32

_NSUB = 16
_EPT = _E // _NSUB
_CH = 400
_NCHUNK = _EPT // _CH
_RPT = 3128
_NPAD = _NSUB * _RPT
_ZR = 136

_BN = 1000
_GRID = _N // _BN


def _edge_body(xr_hbm, src_hbm, dst_hbm, agg_hbm, deg_hbm,
               srcv, dstv, gidxv, rowsv, onesv, zb2, zb1, sem,
               agg_s, deg_s):
  c = lax.axis_index("c")
  s = lax.axis_index("s")

  def _fill2(i, _):
    zb2[i, pl.ds(0, 16)] = jnp.zeros((16,), jnp.float32)
    zb2[i, pl.ds(16, 16)] = jnp.zeros((16,), jnp.float32)
    return _
  lax.fori_loop(0, _ZR, _fill2, 0)

  def _fill1(i, _):
    zb1[pl.ds(i * 16, 16)] = jnp.zeros((16,), jnp.float32)
    return _
  lax.fori_loop(0, 3200 // 16, _fill1, 0)

  def _fillo(i, _):
    onesv[pl.ds(i * 16, 16)] = jnp.full((16,), 1.0, jnp.float32)
    return _
  lax.fori_loop(0, _CH // 16, _fillo, 0)

  def _zero(i, _):
    pltpu.sync_copy(zb2, agg_s.at[pl.ds(s * _RPT + i * _ZR, _ZR)])
    return _
  lax.fori_loop(0, _RPT // _ZR, _zero, 0)

  @pl.when(c == 0)
  def _deg_zero():
    pltpu.sync_copy(zb1.at[pl.ds(0, _RPT)], deg_s.at[pl.ds(s * _RPT, _RPT)])

  plsc.subcore_barrier()

  def _chunk(i, carry):
    base = s * _EPT + i * _CH
    pltpu.sync_copy(src_hbm.at[pl.ds(base, _CH)], srcv)
    pltpu.sync_copy(dst_hbm.at[pl.ds(base, _CH)], dstv)

    def _gidx(j, jcarry):
      sv = srcv[pl.ds(j * 16, 16)]
      gidxv[pl.ds(j * 16, 16)] = sv * 2 + c
      return jcarry
    lax.fori_loop(0, _CH // 16, _gidx, 0)

    pltpu.async_copy(xr_hbm.at[gidxv], rowsv, sem).wait()
    pltpu.sync_copy(rowsv, agg_s.at[dstv], add=True)

    @pl.when(c == 0)
    def _deg_add():
      pltpu.sync_copy(onesv, deg_s.at[dstv], add=True)
    return carry
  lax.fori_loop(0, _NCHUNK, _chunk, 0)

  plsc.subcore_barrier()

  pltpu.sync_copy(agg_s.at[pl.ds(s * _RPT, _RPT)],
                  agg_hbm.at[c, pl.ds(s * _RPT, _RPT)])

  @pl.when(c == 0)
  def _deg_wb():
    pltpu.sync_copy(deg_s.at[pl.ds(s * _RPT, _RPT)],
                    deg_hbm.at[pl.ds(s * _RPT, _RPT)])


_edge_pass = pl.kernel(
    _edge_body,
    out_type=[
        jax.ShapeDtypeStruct((2, _NPAD, 32), jnp.float32),
        jax.ShapeDtypeStruct((_NPAD,), jnp.float32),
    ],
    mesh=plsc.VectorSubcoreMesh(core_axis_name="c", subcore_axis_name="s"),
    compiler_params=pltpu.CompilerParams(use_tc_tiling_on_sc=False),
    scratch_types=[
        pltpu.VMEM((_CH,), jnp.int32),
        pltpu.VMEM((_CH,), jnp.int32),
        pltpu.VMEM((_CH,), jnp.int32),
        pltpu.VMEM((_CH, 32), jnp.float32),
        pltpu.VMEM((_CH,), jnp.float32),
        pltpu.VMEM((_ZR, 32), jnp.float32),
        pltpu.VMEM((3200,), jnp.float32),
        pltpu.SemaphoreType.DMA,
        pltpu.VMEM_SHARED((_NPAD, 32), jnp.float32),
        pltpu.VMEM_SHARED((_NPAD,), jnp.float32),
    ],
)


def _embed_body(sid_ref, cid_ref, semb_ref, cemb_ref, x_ref):
  ohs = (sid_ref[...] == lax.broadcasted_iota(jnp.int32, (_BN, _NS), 1))
  ohc = (cid_ref[...] == lax.broadcasted_iota(jnp.int32, (_BN, _NC), 1))
  x_ref[...] = (
      jnp.dot(ohs.astype(jnp.float32), semb_ref[...],
              preferred_element_type=jnp.float32)
      + jnp.dot(ohc.astype(jnp.float32), cemb_ref[...],
                preferred_element_type=jnp.float32))


def _embed(shape_id, colour_id, shape_emb, col_emb):
  return pl.pallas_call(
      _embed_body,
      grid=(_GRID,),
      in_specs=[
          pl.BlockSpec((_BN, 1), lambda i: (i, 0)),
          pl.BlockSpec((_BN, 1), lambda i: (i, 0)),
          pl.BlockSpec((_NS, _H), lambda i: (0, 0)),
          pl.BlockSpec((_NC, _H), lambda i: (0, 0)),
      ],
      out_specs=pl.BlockSpec((_BN, _H), lambda i: (i, 0)),
      out_shape=jax.ShapeDtypeStruct((_N, _H), jnp.float32),
  )(shape_id.reshape(_N, 1), colour_id.reshape(_N, 1), shape_emb, col_emb)


def _mm_body(aggh_ref, deg_ref, x_ref, wc_ref, b_ref, h_ref, st_ref):
  dinv = 1.0 / jnp.maximum(deg_ref[...], 1.0)
  cat = jnp.concatenate(
      [x_ref[...], aggh_ref[0] * dinv, aggh_ref[1] * dinv], axis=1)
  h = jnp.dot(cat, wc_ref[...], preferred_element_type=jnp.float32) + b_ref[...]
  h_ref[...] = h
  s1 = jnp.sum(h, axis=0, keepdims=True)
  s2 = jnp.sum(h * h, axis=0, keepdims=True)
  st = jnp.concatenate(
      [s1, s2, jnp.zeros((6, _H), jnp.float32)], axis=0)

  @pl.when(pl.program_id(0) == 0)
  def _():
    st_ref[...] = st

  @pl.when(pl.program_id(0) > 0)
  def _():
    st_ref[...] += st


def _mm(aggh, deg, x, wc, b):
  return pl.pallas_call(
      _mm_body,
      grid=(_GRID,),
      in_specs=[
          pl.BlockSpec((2, _BN, 32), lambda i: (0, i, 0)),
          pl.BlockSpec((_BN, 1), lambda i: (i, 0)),
          pl.BlockSpec((_BN, _H), lambda i: (i, 0)),
          pl.BlockSpec((2 * _H, _H), lambda i: (0, 0)),
          pl.BlockSpec((1, _H), lambda i: (0, 0)),
      ],
      out_specs=[
          pl.BlockSpec((_BN, _H), lambda i: (i, 0)),
          pl.BlockSpec((8, _H), lambda i: (0, 0)),
      ],
      out_shape=[
          jax.ShapeDtypeStruct((_N, _H), jnp.float32),
          jax.ShapeDtypeStruct((8, _H), jnp.float32),
      ],
  )(aggh, deg.reshape(_NPAD, 1), x, wc, b.reshape(1, _H))


def _norm_body(h_ref, st_ref, g_ref, b_ref, o_ref):
  st = st_ref[...]
  m = st[0:1] * (1.0 / _N)
  v = st[1:2] * (1.0 / _N) - m * m
  inv = lax.rsqrt(v + 1e-5)
  o_ref[...] = jnp.maximum(
      (h_ref[...] - m) * inv * g_ref[...] + b_ref[...], 0.0)


def _norm(h, st, g, b):
  return pl.pallas_call(
      _norm_body,
      grid=(_GRID,),
      in_specs=[
          pl.BlockSpec((_BN, _H), lambda i: (i, 0)),
          pl.BlockSpec((8, _H), lambda i: (0, 0)),
          pl.BlockSpec((1, _H), lambda i: (0, 0)),
          pl.BlockSpec((1, _H), lambda i: (0, 0)),
      ],
      out_specs=pl.BlockSpec((_BN, _H), lambda i: (i, 0)),
      out_shape=jax.ShapeDtypeStruct((_N, _H), jnp.float32),
  )(h, st, g.reshape(1, _H), b.reshape(1, _H))


def _final_body(h_ref, st_ref, g_ref, b_ref, batch_ref, w_ref, bias_ref, o_ref):
  st = st_ref[...]
  m = st[0:1] * (1.0 / _N)
  v = st[1:2] * (1.0 / _N) - m * m
  inv = lax.rsqrt(v + 1e-5)
  xb = jnp.maximum(
      (h_ref[...] - m) * inv * g_ref[...] + b_ref[...], 0.0)
  oh = (batch_ref[...] == lax.broadcasted_iota(jnp.int32, (_BN, _NG), 1))
  gxp = lax.dot_general(oh.astype(jnp.float32), xb,
                        (((0,), (0,)), ((), ())),
                        preferred_element_type=jnp.float32)
  op = jnp.dot(gxp, w_ref[...], preferred_element_type=jnp.float32)

  @pl.when(pl.program_id(0) == 0)
  def _():
    o_ref[...] = op + bias_ref[...]

  @pl.when(pl.program_id(0) > 0)
  def _():
    o_ref[...] += op


def _final(h, st, g, b, batch, w_pad, bias_pad):
  return pl.pallas_call(
      _final_body,
      grid=(_GRID,),
      in_specs=[
          pl.BlockSpec((_BN, _H), lambda i: (i, 0)),
          pl.BlockSpec((8, _H), lambda i: (0, 0)),
          pl.BlockSpec((1, _H), lambda i: (0, 0)),
          pl.BlockSpec((1, _H), lambda i: (0, 0)),
          pl.BlockSpec((_BN, 1), lambda i: (i, 0)),
          pl.BlockSpec((_H, 128), lambda i: (0, 0)),
          pl.BlockSpec((1, 128), lambda i: (0, 0)),
      ],
      out_specs=pl.BlockSpec((_NG, 128), lambda i: (0, 0)),
      out_shape=jax.ShapeDtypeStruct((_NG, 128), jnp.float32),
  )(h, st, g.reshape(1, _H), b.reshape(1, _H), batch.reshape(_N, 1),
    w_pad, bias_pad)


def kernel(shape_id, colour_id, edge_index, batch, shape_emb, col_emb,
           W1l, b1l, W1r, bn1_g, bn1_b, W2l, b2l, W2r, bn2_g, bn2_b,
           lin_W, lin_b):
  src = edge_index[0]
  dst = edge_index[1]
  wc1 = jnp.concatenate([W1r, W1l], axis=0)
  wc2 = jnp.concatenate([W2r, W2l], axis=0)
  w_pad = jnp.pad(lin_W, ((0, 0), (0, 128 - lin_W.shape[1])))
  bias_pad = jnp.pad(lin_b, (0, 128 - lin_b.shape[0])).reshape(1, 128)

  x = _embed(shape_id, colour_id, shape_emb, col_emb)
  agg1, deg = _edge_pass(x.reshape(2 * _N, 32), src, dst)
  h1, st1 = _mm(agg1, deg, x, wc1, b1l)
  x1 = _norm(h1, st1, bn1_g, bn1_b)
  agg2, _ = _edge_pass(x1.reshape(2 * _N, 32), src, dst)
  h2, st2 = _mm(agg2, deg, x1, wc2, b2l)
  out = _final(h2, st2, bn2_g, bn2_b, batch, w_pad, bias_pad)
  return out[:, : lin_b.shape[0]]


# speedup vs baseline: 8.2984x; 1.2822x over previous
import functools

import jax
import jax.numpy as jnp
from jax import lax
from jax.experimental import pallas as pl
from jax.experimental.pallas import tpu as pltpu
from jax.experimental.pallas import tpu_sc as plsc

_N = 50000
_E = 800000
_H = 64
_NG = 512
_NS = 64
_NC = 32

_NSUB = 16
_EPT = _E // _NSUB
_CH = 400
_NCHUNK = _EPT // _CH
_RPT = 3128
_NPAD = _NSUB * _RPT
_ZR = 136

_BN = 1000
_GRID = _N // _BN


def _edge_body(xr_hbm, src_hbm, dst_hbm, zdeg_hbm, agg_hbm, deg_hbm,
               src0, src1, dst0, dst1, rows0, rows1, onesv,
               sg0, sg1, ss0, ss1, sd0, sd1,
               agg_s, deg_s):
  c = lax.axis_index("c")
  s = lax.axis_index("s")
  srcb = (src0, src1)
  dstb = (dst0, dst1)
  rowsb = (rows0, rows1)
  sg = (sg0, sg1)
  ss = (ss0, ss1)
  sd = (sd0, sd1)

  def _fillo(i, icarry):
    onesv[pl.ds(i * 16, 16)] = jnp.full((16,), 1.0, jnp.float32)
    return icarry
  lax.fori_loop(0, _CH // 16, _fillo, 0)

  def _fillz(i, icarry):
    rows0[i, pl.ds(0, 16)] = jnp.zeros((16,), jnp.float32)
    rows0[i, pl.ds(16, 16)] = jnp.zeros((16,), jnp.float32)
    return icarry
  lax.fori_loop(0, _CH, _fillz, 0)

  for jj in range(_RPT // _CH):
    pltpu.sync_copy(rows0, agg_s.at[pl.ds(s * _RPT + jj * _CH, _CH)])
  _TAIL = _RPT - (_RPT // _CH) * _CH
  pltpu.sync_copy(rows0.at[pl.ds(0, _TAIL)],
                  agg_s.at[pl.ds(s * _RPT + (_RPT // _CH) * _CH, _TAIL)])

  @pl.when(c == 0)
  def _deg_zero():
    pltpu.sync_copy(zdeg_hbm, deg_s.at[pl.ds(s * _RPT, _RPT)])

  plsc.subcore_barrier()

  def _load_and_fire(k, p):
    base = s * _EPT + k * _CH
    pltpu.sync_copy(src_hbm.at[pl.ds(base, _CH)], srcb[p])
    pltpu.sync_copy(dst_hbm.at[pl.ds(base, _CH)], dstb[p])

    def _gidx(j, jcarry):
      sv = srcb[p][pl.ds(j * 16, 16)]
      srcb[p][pl.ds(j * 16, 16)] = sv * 2 + c
      return jcarry
    lax.fori_loop(0, _CH // 16, _gidx, 0)
    pltpu.async_copy(xr_hbm.at[srcb[p]], rowsb[p], sg[p])

  _load_and_fire(0, 0)

  def _outer(i, carry):
    for j in range(4):
      k = 4 * i + j
      p = j % 2
      q = 1 - p

      @pl.when(k < _NCHUNK)
      def _body():
        @pl.when(k + 1 < _NCHUNK)
        def _prep():
          @pl.when(k > 0)
          def _wait_prev():
            pltpu.make_async_copy(rowsb[q], agg_s.at[dstb[q]], ss[q]).wait()

            @pl.when(c == 0)
            def _wait_deg():
              pltpu.make_async_copy(onesv, deg_s.at[dstb[q]], sd[q]).wait()
          _load_and_fire(k + 1, q)

        pltpu.make_async_copy(xr_hbm.at[srcb[p]], rowsb[p], sg[p]).wait()
        pltpu.async_copy(rowsb[p], agg_s.at[dstb[p]], ss[p], add=True)

        @pl.when(c == 0)
        def _deg_add():
          pltpu.async_copy(onesv, deg_s.at[dstb[p]], sd[p], add=True)
    return carry
  lax.fori_loop(0, (_NCHUNK + 3) // 4, _outer, 0)

  for p in range(2):
    pltpu.make_async_copy(rowsb[p], agg_s.at[dstb[p]], ss[p]).wait()

    @pl.when(c == 0)
    def _drain_deg():
      pltpu.make_async_copy(onesv, deg_s.at[dstb[p]], sd[p]).wait()

  plsc.subcore_barrier()

  pltpu.sync_copy(agg_s.at[pl.ds(s * _RPT, _RPT)],
                  agg_hbm.at[c, pl.ds(s * _RPT, _RPT)])

  @pl.when(c == 0)
  def _deg_wb():
    pltpu.sync_copy(deg_s.at[pl.ds(s * _RPT, _RPT)],
                    deg_hbm.at[pl.ds(s * _RPT, _RPT)])


_edge_pass = pl.kernel(
    _edge_body,
    out_type=[
        jax.ShapeDtypeStruct((2, _NPAD, 32), jnp.float32),
        jax.ShapeDtypeStruct((_NPAD,), jnp.float32),
    ],
    mesh=plsc.VectorSubcoreMesh(core_axis_name="c", subcore_axis_name="s"),
    compiler_params=pltpu.CompilerParams(use_tc_tiling_on_sc=False),
    scratch_types=[
        pltpu.VMEM((_CH,), jnp.int32),
        pltpu.VMEM((_CH,), jnp.int32),
        pltpu.VMEM((_CH,), jnp.int32),
        pltpu.VMEM((_CH,), jnp.int32),
        pltpu.VMEM((_CH, 32), jnp.float32),
        pltpu.VMEM((_CH, 32), jnp.float32),
        pltpu.VMEM((_CH,), jnp.float32),
        pltpu.SemaphoreType.DMA,
        pltpu.SemaphoreType.DMA,
        pltpu.SemaphoreType.DMA,
        pltpu.SemaphoreType.DMA,
        pltpu.SemaphoreType.DMA,
        pltpu.SemaphoreType.DMA,
        pltpu.VMEM_SHARED((_NPAD, 32), jnp.float32),
        pltpu.VMEM_SHARED((_NPAD,), jnp.float32),
    ],
)


def _embed_body(sid_ref, cid_ref, semb_ref, cemb_ref, x_ref):
  ohs = (sid_ref[...] == lax.broadcasted_iota(jnp.int32, (_BN, _NS), 1))
  ohc = (cid_ref[...] == lax.broadcasted_iota(jnp.int32, (_BN, _NC), 1))
  x_ref[...] = (
      jnp.dot(ohs.astype(jnp.float32), semb_ref[...],
              preferred_element_type=jnp.float32)
      + jnp.dot(ohc.astype(jnp.float32), cemb_ref[...],
                preferred_element_type=jnp.float32))


def _embed(shape_id, colour_id, shape_emb, col_emb):
  return pl.pallas_call(
      _embed_body,
      grid=(_GRID,),
      in_specs=[
          pl.BlockSpec((_BN, 1), lambda i: (i, 0)),
          pl.BlockSpec((_BN, 1), lambda i: (i, 0)),
          pl.BlockSpec((_NS, _H), lambda i: (0, 0)),
          pl.BlockSpec((_NC, _H), lambda i: (0, 0)),
      ],
      out_specs=pl.BlockSpec((_BN, _H), lambda i: (i, 0)),
      out_shape=jax.ShapeDtypeStruct((_N, _H), jnp.float32),
  )(shape_id.reshape(_N, 1), colour_id.reshape(_N, 1), shape_emb, col_emb)


def _mm_body(aggh_ref, deg_ref, x_ref, wc_ref, b_ref, h_ref, st_ref):
  dinv = 1.0 / jnp.maximum(deg_ref[...], 1.0)
  cat = jnp.concatenate(
      [x_ref[...], aggh_ref[0] * dinv, aggh_ref[1] * dinv], axis=1)
  h = jnp.dot(cat, wc_ref[...], preferred_element_type=jnp.float32) + b_ref[...]
  h_ref[...] = h
  s1 = jnp.sum(h, axis=0, keepdims=True)
  s2 = jnp.sum(h * h, axis=0, keepdims=True)
  st = jnp.concatenate(
      [s1, s2, jnp.zeros((6, _H), jnp.float32)], axis=0)

  @pl.when(pl.program_id(0) == 0)
  def _():
    st_ref[...] = st

  @pl.when(pl.program_id(0) > 0)
  def _():
    st_ref[...] += st


def _mm(aggh, deg, x, wc, b):
  return pl.pallas_call(
      _mm_body,
      grid=(_GRID,),
      in_specs=[
          pl.BlockSpec((2, _BN, 32), lambda i: (0, i, 0)),
          pl.BlockSpec((_BN, 1), lambda i: (i, 0)),
          pl.BlockSpec((_BN, _H), lambda i: (i, 0)),
          pl.BlockSpec((2 * _H, _H), lambda i: (0, 0)),
          pl.BlockSpec((1, _H), lambda i: (0, 0)),
      ],
      out_specs=[
          pl.BlockSpec((_BN, _H), lambda i: (i, 0)),
          pl.BlockSpec((8, _H), lambda i: (0, 0)),
      ],
      out_shape=[
          jax.ShapeDtypeStruct((_N, _H), jnp.float32),
          jax.ShapeDtypeStruct((8, _H), jnp.float32),
      ],
  )(aggh, deg.reshape(_NPAD, 1), x, wc, b.reshape(1, _H))


def _norm_body(h_ref, st_ref, g_ref, b_ref, o_ref):
  st = st_ref[...]
  m = st[0:1] * (1.0 / _N)
  v = st[1:2] * (1.0 / _N) - m * m
  inv = lax.rsqrt(v + 1e-5)
  o_ref[...] = jnp.maximum(
      (h_ref[...] - m) * inv * g_ref[...] + b_ref[...], 0.0)


def _norm(h, st, g, b):
  return pl.pallas_call(
      _norm_body,
      grid=(_GRID,),
      in_specs=[
          pl.BlockSpec((_BN, _H), lambda i: (i, 0)),
          pl.BlockSpec((8, _H), lambda i: (0, 0)),
          pl.BlockSpec((1, _H), lambda i: (0, 0)),
          pl.BlockSpec((1, _H), lambda i: (0, 0)),
      ],
      out_specs=pl.BlockSpec((_BN, _H), lambda i: (i, 0)),
      out_shape=jax.ShapeDtypeStruct((_N, _H), jnp.float32),
  )(h, st, g.reshape(1, _H), b.reshape(1, _H))


def _final_body(h_ref, st_ref, g_ref, b_ref, batch_ref, w_ref, bias_ref, o_ref):
  st = st_ref[...]
  m = st[0:1] * (1.0 / _N)
  v = st[1:2] * (1.0 / _N) - m * m
  inv = lax.rsqrt(v + 1e-5)
  xb = jnp.maximum(
      (h_ref[...] - m) * inv * g_ref[...] + b_ref[...], 0.0)
  oh = (batch_ref[...] == lax.broadcasted_iota(jnp.int32, (_BN, _NG), 1))
  gxp = lax.dot_general(oh.astype(jnp.float32), xb,
                        (((0,), (0,)), ((), ())),
                        preferred_element_type=jnp.float32)
  op = jnp.dot(gxp, w_ref[...], preferred_element_type=jnp.float32)

  @pl.when(pl.program_id(0) == 0)
  def _():
    o_ref[...] = op + bias_ref[...]

  @pl.when(pl.program_id(0) > 0)
  def _():
    o_ref[...] += op


def _final(h, st, g, b, batch, w_pad, bias_pad):
  return pl.pallas_call(
      _final_body,
      grid=(_GRID,),
      in_specs=[
          pl.BlockSpec((_BN, _H), lambda i: (i, 0)),
          pl.BlockSpec((8, _H), lambda i: (0, 0)),
          pl.BlockSpec((1, _H), lambda i: (0, 0)),
          pl.BlockSpec((1, _H), lambda i: (0, 0)),
          pl.BlockSpec((_BN, 1), lambda i: (i, 0)),
          pl.BlockSpec((_H, 128), lambda i: (0, 0)),
          pl.BlockSpec((1, 128), lambda i: (0, 0)),
      ],
      out_specs=pl.BlockSpec((_NG, 128), lambda i: (0, 0)),
      out_shape=jax.ShapeDtypeStruct((_NG, 128), jnp.float32),
  )(h, st, g.reshape(1, _H), b.reshape(1, _H), batch.reshape(_N, 1),
    w_pad, bias_pad)


def kernel(shape_id, colour_id, edge_index, batch, shape_emb, col_emb,
           W1l, b1l, W1r, bn1_g, bn1_b, W2l, b2l, W2r, bn2_g, bn2_b,
           lin_W, lin_b):
  src = edge_index[0]
  dst = edge_index[1]
  wc1 = jnp.concatenate([W1r, W1l], axis=0)
  wc2 = jnp.concatenate([W2r, W2l], axis=0)
  w_pad = jnp.pad(lin_W, ((0, 0), (0, 128 - lin_W.shape[1])))
  bias_pad = jnp.pad(lin_b, (0, 128 - lin_b.shape[0])).reshape(1, 128)

  x = _embed(shape_id, colour_id, shape_emb, col_emb)
  zdeg = jnp.zeros((_RPT,), jnp.float32)
  agg1, deg = _edge_pass(x.reshape(2 * _N, 32), src, dst, zdeg)
  h1, st1 = _mm(agg1, deg, x, wc1, b1l)
  x1 = _norm(h1, st1, bn1_g, bn1_b)
  agg2, _ = _edge_pass(x1.reshape(2 * _N, 32), src, dst, zdeg)
  h2, st2 = _mm(agg2, deg, x1, wc2, b2l)
  out = _final(h2, st2, bn2_g, bn2_b, batch, w_pad, bias_pad)
  return out[:, : lin_b.shape[0]]
